# Initial kernel scaffold; baseline (speedup 1.0000x reference)
import jax
import jax.numpy as jnp
from jax.experimental import pallas as pl


def kernel(x, edge_index, edge_attr, batch, target_data, W1, b1, W2, b2, conv_w, conv_b, cfc_w, cfc_b, out_w, out_b):
    raise NotImplementedError("write your pallas kernel here")


# speedup vs baseline: 8.6343x; 8.6343x over previous
import functools

import jax
import jax.numpy as jnp
from jax import lax
from jax.experimental import pallas as pl
from jax.experimental.pallas import tpu as pltpu
from jax.experimental.pallas import tpu_sc as plsc

N = 100000
E = 3200000
G = 1024

NC = 2
NS = 16
HALF = N // NC
GSPREAD = 512
ROWS = HALF + GSPREAD
XP = 32
AE = 8

ZROWS = 3200
NBLK = 98
NPAD = NBLK * 1024

CHX = 512
NCHX = E // CHX
FKX = NCHX // NS
EXX = NCHX - FKX * NS

CHA = 1024
NCHA = E // CHA
FKA = NCHA // NS
EXA = NCHA - FKA * NS


def _local_idx(dstv, lidx, cbase, iota16, nrows):
  for j in range(nrows):
    for t in range(8):
      d = dstv[j, pl.ds(t * 16, 16)]
      ld = d - cbase
      ok = (ld >= 0) & (ld < HALF)
      garb = HALF + ((j * 128 + t * 16 + iota16) & (GSPREAD - 1))
      lidx[j, pl.ds(t * 16, 16)] = jnp.where(ok, ld, garb)


def _zero_init(z_hbm, agg_sh, s):
  @pl.when(s < NS - 1)
  def _():
    pltpu.sync_copy(z_hbm.at[pl.ds(0, ZROWS)],
                    agg_sh.at[pl.ds(s * ZROWS, ZROWS)])

  @pl.when(s == NS - 1)
  def _():
    n = ROWS - (NS - 1) * ZROWS
    pltpu.sync_copy(z_hbm.at[pl.ds(0, n)],
                    agg_sh.at[pl.ds((NS - 1) * ZROWS, n)])


def _copy_out(agg_sh, agg_out, s, cbase):
  @pl.when(s < NS - 1)
  def _():
    pltpu.sync_copy(agg_sh.at[pl.ds(s * ZROWS, ZROWS)],
                    agg_out.at[pl.ds(cbase + s * ZROWS, ZROWS)])

  @pl.when(s == NS - 1)
  def _():
    n = HALF - (NS - 1) * ZROWS
    pltpu.sync_copy(agg_sh.at[pl.ds((NS - 1) * ZROWS, n)],
                    agg_out.at[pl.ds(cbase + (NS - 1) * ZROWS, n)])


def _sc_x_body(src2d, dst2d, xpad_hbm, zx_hbm, aggx_out,
               aggx_sh, srcv, dstv, lidx, xrows, gsem, ssem, csem):
  c = lax.axis_index("c")
  s = lax.axis_index("s")
  cbase = c * HALF

  _zero_init(zx_hbm, aggx_sh, s)
  plsc.subcore_barrier()

  iota16 = lax.iota(jnp.int32, 16)

  def do_chunk(q):
    d1 = pltpu.async_copy(src2d.at[pl.ds(q * 4, 4)], srcv, csem)
    d2 = pltpu.async_copy(dst2d.at[pl.ds(q * 4, 4)], dstv, csem)
    d1.wait()
    d2.wait()
    _local_idx(dstv, lidx, cbase, iota16, 4)
    gd = [pltpu.async_copy(xpad_hbm.at[srcv.at[j]],
                           xrows.at[pl.ds(j * 128, 128)], gsem)
          for j in range(4)]
    for d in gd:
      d.wait()
    sd = [pltpu.async_copy(xrows.at[pl.ds(j * 128, 128)],
                           aggx_sh.at[lidx.at[j]], ssem, add=True)
          for j in range(4)]
    for d in sd:
      d.wait()

  def loop_body(k, carry):
    do_chunk(k * NS + s)
    return carry

  lax.fori_loop(0, FKX, loop_body, 0)

  @pl.when(s < EXX)
  def _():
    do_chunk(FKX * NS + s)

  plsc.subcore_barrier()
  _copy_out(aggx_sh, aggx_out, s, cbase)


def _sc_attr_body(dst2d, attr8, za_hbm, agge_out,
                  agge_sh, dstv, lidx, attrv, ssem, csem):
  c = lax.axis_index("c")
  s = lax.axis_index("s")
  cbase = c * HALF

  _zero_init(za_hbm, agge_sh, s)
  plsc.subcore_barrier()

  iota16 = lax.iota(jnp.int32, 16)

  def do_chunk(q):
    d2 = pltpu.async_copy(dst2d.at[pl.ds(q * 8, 8)], dstv, csem)
    d3 = pltpu.async_copy(attr8.at[pl.ds(q * CHA, CHA)], attrv, csem)
    d2.wait()
    d3.wait()
    _local_idx(dstv, lidx, cbase, iota16, 8)
    sd = [pltpu.async_copy(attrv.at[pl.ds(j * 128, 128)],
                           agge_sh.at[lidx.at[j]], ssem, add=True)
          for j in range(8)]
    for d in sd:
      d.wait()

  def loop_body(k, carry):
    do_chunk(k * NS + s)
    return carry

  lax.fori_loop(0, FKA, loop_body, 0)

  @pl.when(s < EXA)
  def _():
    do_chunk(FKA * NS + s)

  plsc.subcore_barrier()
  _copy_out(agge_sh, agge_out, s, cbase)


@functools.cache
def _get_sc_x():
  return pl.kernel(
      _sc_x_body,
      out_type=jax.ShapeDtypeStruct((NPAD, XP), jnp.float32),
      mesh=plsc.VectorSubcoreMesh(core_axis_name="c", subcore_axis_name="s"),
      compiler_params=pltpu.CompilerParams(use_tc_tiling_on_sc=False),
      scratch_types=[
          pltpu.VMEM_SHARED((ROWS, XP), jnp.float32),
          pltpu.VMEM((4, 128), jnp.int32),
          pltpu.VMEM((4, 128), jnp.int32),
          pltpu.VMEM((4, 128), jnp.int32),
          pltpu.VMEM((CHX, XP), jnp.float32),
          pltpu.SemaphoreType.DMA,
          pltpu.SemaphoreType.DMA,
          pltpu.SemaphoreType.DMA,
      ],
  )


@functools.cache
def _get_sc_attr():
  return pl.kernel(
      _sc_attr_body,
      out_type=jax.ShapeDtypeStruct((NPAD, AE), jnp.float32),
      mesh=plsc.VectorSubcoreMesh(core_axis_name="c", subcore_axis_name="s"),
      compiler_params=pltpu.CompilerParams(use_tc_tiling_on_sc=False),
      scratch_types=[
          pltpu.VMEM_SHARED((ROWS, AE), jnp.float32),
          pltpu.VMEM((8, 128), jnp.int32),
          pltpu.VMEM((8, 128), jnp.int32),
          pltpu.VMEM((CHA, AE), jnp.float32),
          pltpu.SemaphoreType.DMA,
          pltpu.SemaphoreType.DMA,
      ],
  )


def _tc_body(aggx, agge, batch3, tdf, w1a, w1b, b1r, w2, b2r,
             wflat, cbr, cfcw, cfbr, owd, owt, obr,
             out, pooled, cnt):
  i = pl.program_id(0)

  @pl.when(i == 0)
  def _():
    pooled[...] = jnp.zeros_like(pooled)
    cnt[...] = jnp.zeros_like(cnt)

  r = aggx[...] @ w1a[...] + agge[...] @ w1b[...] + b1r[...]
  r = jnp.maximum(r, 0.0)
  col = lax.broadcasted_iota(jnp.int32, (1024, 1), 0) + i * 1024
  r = jnp.where(col < N, r, 0.0)
  brow = batch3[0]
  mrow = (lax.broadcasted_iota(jnp.int32, (1, 1024), 1) + i * 1024) < N
  gi = lax.broadcasted_iota(jnp.int32, (1024, 1024), 0)
  oh = ((gi == brow) & mrow).astype(jnp.float32)
  pooled[...] += jnp.dot(oh, r, preferred_element_type=jnp.float32)
  cnt[...] += jnp.dot(oh, jnp.ones((1024, 8), jnp.float32),
                      preferred_element_type=jnp.float32)

  @pl.when(i == NBLK - 1)
  def _():
    c = jnp.maximum(cnt[:, 0:1], 1.0)
    drug = (pooled[...] / c) @ w2[...] + b2r[...]
    t0 = jnp.dot(tdf[:, 0:105], wflat[...],
                 preferred_element_type=jnp.float32)
    t1 = jnp.dot(tdf[:, 21:126], wflat[...],
                 preferred_element_type=jnp.float32)
    cc = jnp.maximum(jnp.maximum(t0, t1) + cbr[...], 0.0)
    tf = jnp.dot(cc, cfcw[...], preferred_element_type=jnp.float32) + cfbr[...]
    z = (jnp.dot(drug, owd[...], preferred_element_type=jnp.float32)
         + jnp.dot(tf, owt[...], preferred_element_type=jnp.float32)
         + obr[0, 0])
    out[...] = 1.0 / (1.0 + jnp.exp(-z))


def _tc_head(aggx, agge, batch3, tdf, w1a, w1b, b1r, w2, b2r,
             wflat, cbr, cfcw, cfbr, owd, owt, obr):
  whole = lambda *shape: pl.BlockSpec(shape, lambda i: tuple(0 for _ in shape))
  return pl.pallas_call(
      _tc_body,
      grid=(NBLK,),
      in_specs=[
          pl.BlockSpec((1024, XP), lambda i: (i, 0)),
          pl.BlockSpec((1024, AE), lambda i: (i, 0)),
          pl.BlockSpec((1, 1, 1024), lambda i: (i, 0, 0)),
          whole(1024, 126),
          whole(XP, 128),
          whole(AE, 128),
          whole(1, 128),
          whole(128, 256),
          whole(1, 256),
          whole(105, 64),
          whole(1, 64),
          whole(64, 128),
          whole(1, 128),
          whole(256, 1),
          whole(128, 1),
          whole(1, 1),
      ],
      out_specs=pl.BlockSpec((1024, 1), lambda i: (0, 0)),
      out_shape=jax.ShapeDtypeStruct((G, 1), jnp.float32),
      scratch_shapes=[
          pltpu.VMEM((1024, 128), jnp.float32),
          pltpu.VMEM((1024, 8), jnp.float32),
      ],
  )(aggx, agge, batch3, tdf, w1a, w1b, b1r, w2, b2r,
    wflat, cbr, cfcw, cfbr, owd, owt, obr)


@jax.jit
def kernel(x, edge_index, edge_attr, batch, target_data,
           W1, b1, W2, b2, conv_w, conv_b, cfc_w, cfc_b, out_w, out_b):
  src2d = edge_index[0].reshape(E // 128, 128)
  dst2d = edge_index[1].reshape(E // 128, 128)
  xpad = jnp.concatenate(
      [x, jnp.zeros((N, XP - x.shape[1]), jnp.float32)], axis=1)
  attr8 = jnp.concatenate(
      [edge_attr, jnp.zeros((E, AE - edge_attr.shape[1]), jnp.float32)],
      axis=1)
  zx = jnp.zeros((ZROWS, XP), jnp.float32)
  za = jnp.zeros((ZROWS, AE), jnp.float32)

  aggx = _get_sc_x()(src2d, dst2d, xpad, zx)
  agge = _get_sc_attr()(dst2d, attr8, za)

  batch3 = jnp.pad(batch, (0, NPAD - N)).reshape(NBLK, 1, 1024)
  tdf = target_data.transpose(0, 2, 1).reshape(G, 126)
  w1a = jnp.concatenate(
      [W1[:20], jnp.zeros((XP - 20, 128), jnp.float32)], axis=0)
  w1b = jnp.concatenate(
      [W1[20:25], jnp.zeros((AE - 5, 128), jnp.float32)], axis=0)
  wflat = conv_w.transpose(2, 1, 0).reshape(105, 64)
  return _tc_head(
      aggx, agge, batch3, tdf, w1a, w1b, b1.reshape(1, 128),
      W2, b2.reshape(1, 256), wflat, conv_b.reshape(1, 64),
      cfc_w, cfc_b.reshape(1, 128), out_w[:256], out_w[256:],
      out_b.reshape(1, 1))
